# baseline (speedup 1.0000x reference)
import functools

import jax
import jax.numpy as jnp
from jax import lax
from jax.experimental import pallas as pl
from jax.experimental.pallas import tpu as pltpu
from jax.experimental.pallas import tpu_sc as plsc

BATCH = 16384
HIST = 50
VOCAB = 1000000
DIM = 32

N = BATCH * HIST
NC = 2
NS = 16
NW = NC * NS
PER_W = N // NW
CHUNK = 1280
NCHUNK = PER_W // CHUNK
GCH = 1
GROUP = GCH * CHUNK
NBUF = 2
NGRP = NCHUNK // GCH
NGG = NGRP // NBUF

_MESH = plsc.VectorSubcoreMesh(core_axis_name="c", subcore_axis_name="s")


@functools.partial(
    pl.kernel,
    mesh=_MESH,
    compiler_params=pltpu.CompilerParams(use_tc_tiling_on_sc=False),
    out_type=jax.ShapeDtypeStruct((N, DIM), jnp.float32),
    scratch_types=[
        pltpu.VMEM((NCHUNK, CHUNK), jnp.int32),
        pltpu.VMEM((NBUF, GROUP, DIM), jnp.float32),
        pltpu.SemaphoreType.DMA,
        pltpu.SemaphoreType.DMA((NBUF,)),
    ],
)
def _embed_sc(idx_hbm, table_hbm, out_hbm, idx_v, rows_v, gsem, ssem):
    wid = lax.axis_index("s") * NC + lax.axis_index("c")
    base = wid * PER_W
    pltpu.sync_copy(idx_hbm.at[wid], idx_v)

    def group_body(gg, _):
        for b in range(NBUF):
            g = gg * NBUF + b

            @pl.when(gg >= 1)
            def _():
                pltpu.make_async_copy(
                    rows_v.at[b], out_hbm.at[pl.ds(base, GROUP)], ssem.at[b]
                ).wait()

            handles = [
                pltpu.async_copy(
                    table_hbm.at[idx_v.at[g * GCH + k]],
                    rows_v.at[b, pl.ds(k * CHUNK, CHUNK)],
                    gsem,
                )
                for k in range(GCH)
            ]
            for h in handles:
                h.wait()

            pltpu.async_copy(
                rows_v.at[b], out_hbm.at[pl.ds(base + g * GROUP, GROUP)], ssem.at[b]
            )
        return 0

    lax.fori_loop(0, NGG, group_body, 0)

    for b in range(NBUF):
        pltpu.make_async_copy(
            rows_v.at[b], out_hbm.at[pl.ds(base, GROUP)], ssem.at[b]
        ).wait()


def kernel(inputs, embeddings):
    idx = inputs.astype(jnp.int32).reshape(NW, NCHUNK, CHUNK)
    out = _embed_sc(idx, embeddings)
    return out.reshape(BATCH, HIST, DIM)


# speedup vs baseline: 1.3961x; 1.3961x over previous
import functools

import jax
import jax.numpy as jnp
from jax import lax
from jax.experimental import pallas as pl
from jax.experimental.pallas import tpu as pltpu
from jax.experimental.pallas import tpu_sc as plsc

BATCH = 16384
HIST = 50
VOCAB = 1000000
DIM = 32

NC = 2
NS = 16
NW = NC * NS
BPW = BATCH // NW
BCH = 128
NBCH = BPW // BCH
NBLK = HIST * NBCH

_MESH = plsc.VectorSubcoreMesh(core_axis_name="c", subcore_axis_name="s")


@functools.partial(
    pl.kernel,
    mesh=_MESH,
    compiler_params=pltpu.CompilerParams(needs_layout_passes=False),
    out_type=jax.ShapeDtypeStruct((HIST, DIM, BATCH), jnp.float32),
    scratch_types=[
        pltpu.VMEM((HIST, BPW), jnp.int32),
        pltpu.VMEM((2, BCH), jnp.int32),
        pltpu.VMEM((2, BCH, 128), jnp.float32),
        pltpu.VMEM((2, DIM, BCH), jnp.float32),
        pltpu.SemaphoreType.DMA,
        pltpu.SemaphoreType.DMA((2,)),
    ],
)
def _embed_sc(idx_hbm, packed_hbm, out_hbm, idx_v, sidx_v, grp_v, trb_v, gsem, ssem):
    wid = lax.axis_index("s") * NC + lax.axis_index("c")
    b0 = wid * BPW
    pltpu.sync_copy(idx_hbm.at[:, pl.ds(b0, BPW)], idx_v)

    iota16 = lax.iota(jnp.int32, 16)

    def prep_and_fire(t, buf):
        h = t // NBCH
        coff = (t % NBCH) * BCH
        for jv in range(BCH // 16):
            iv = idx_v[h, pl.ds(coff + jv * 16, 16)]
            sidx_v[buf, pl.ds(jv * 16, 16)] = lax.shift_right_logical(iv, 2)
        pltpu.async_copy(packed_hbm.at[sidx_v.at[buf]], grp_v.at[buf], gsem)

    def extract(t, buf):
        h = t // NBCH
        coff = (t % NBCH) * BCH
        for jv in range(BCH // 16):
            iv = idx_v[h, pl.ds(coff + jv * 16, 16)]
            colb = lax.rem(iv, 4) * 32
            rows = iota16 + jv * 16
            for d in range(DIM):
                vals = plsc.load_gather(grp_v.at[buf], [rows, colb + d])
                trb_v[buf, d, pl.ds(jv * 16, 16)] = vals

    def write_out(t, buf):
        h = t // NBCH
        bcol = b0 + (t % NBCH) * BCH
        pltpu.async_copy(
            trb_v.at[buf], out_hbm.at[h, :, pl.ds(bcol, BCH)], ssem.at[buf]
        )

    prep_and_fire(0, 0)

    def body(t2, _):
        for k in range(2):
            t = t2 * 2 + k
            buf = k
            if k == 0:
                prep_and_fire(t + 1, 1 - buf)
            else:
                @pl.when(t2 < NBLK // 2 - 1)
                def _():
                    prep_and_fire(t + 1, 1 - buf)
            pltpu.make_async_copy(
                packed_hbm.at[sidx_v.at[buf]], grp_v.at[buf], gsem
            ).wait()
            @pl.when(t2 >= 1)
            def _():
                pltpu.make_async_copy(
                    trb_v.at[buf], out_hbm.at[0, :, pl.ds(0, BCH)], ssem.at[buf]
                ).wait()
            extract(t, buf)
            write_out(t, buf)
        return 0

    lax.fori_loop(0, NBLK // 2, body, 0)

    for buf in range(2):
        pltpu.make_async_copy(
            trb_v.at[buf], out_hbm.at[0, :, pl.ds(0, BCH)], ssem.at[buf]
        ).wait()


def kernel(inputs, embeddings):
    packed = embeddings.reshape(VOCAB // 4, 128)
    idx_t = inputs.astype(jnp.int32).T
    out = _embed_sc(idx_t, packed)
    return out.transpose(2, 0, 1)


# speedup vs baseline: 1.8697x; 1.3392x over previous
import functools

import jax
import jax.numpy as jnp
from jax import lax
from jax.experimental import pallas as pl
from jax.experimental.pallas import tpu as pltpu
from jax.experimental.pallas import tpu_sc as plsc

BATCH = 16384
HIST = 50
VOCAB = 1000000
DIM = 32

NC = 2
NS = 16
NW = NC * NS
BPW = BATCH // NW
BCH = 128
NBCH = BPW // BCH
NBLK = HIST * NBCH

_MESH = plsc.VectorSubcoreMesh(core_axis_name="c", subcore_axis_name="s")


@functools.partial(
    pl.kernel,
    mesh=_MESH,
    compiler_params=pltpu.CompilerParams(needs_layout_passes=False),
    out_type=jax.ShapeDtypeStruct((HIST, DIM, BATCH), jnp.float32),
    scratch_types=[
        pltpu.VMEM((HIST, BPW), jnp.int32),
        pltpu.VMEM((2, BCH), jnp.int32),
        pltpu.VMEM((2, BCH, 128), jnp.float32),
        pltpu.VMEM((2, DIM, BCH), jnp.float32),
        pltpu.SemaphoreType.DMA,
        pltpu.SemaphoreType.DMA((2,)),
    ],
)
def _embed_sc(idx_hbm, packed_hbm, out_hbm, idx_v, sidx_v, grp_v, trb_v, gsem, ssem):
    wid = lax.axis_index("s") * NC + lax.axis_index("c")
    b0 = wid * BPW
    pltpu.sync_copy(idx_hbm.at[:, pl.ds(b0, BPW)], idx_v)

    iota16 = lax.iota(jnp.int32, 16)

    def prep_and_fire(t, buf):
        h = t // NBCH
        coff = (t % NBCH) * BCH
        for jv in range(BCH // 16):
            iv = idx_v[h, pl.ds(coff + jv * 16, 16)]
            sidx_v[buf, pl.ds(jv * 16, 16)] = lax.shift_right_logical(iv, 2)
        pltpu.async_copy(packed_hbm.at[sidx_v.at[buf]], grp_v.at[buf], gsem)

    def extract(t, buf):
        h = t // NBCH
        coff = (t % NBCH) * BCH
        for jv in range(BCH // 16):
            iv = idx_v[h, pl.ds(coff + jv * 16, 16)]
            colb = lax.shift_left(jnp.bitwise_and(iv, 3), 5)
            rows = iota16 + jv * 16
            for d8 in range(0, DIM, 8):
                vals = [
                    plsc.load_gather(grp_v.at[buf], [rows, colb + (d8 + i)])
                    for i in range(8)
                ]
                for i in range(8):
                    trb_v[buf, d8 + i, pl.ds(jv * 16, 16)] = vals[i]

    def write_out(t, buf):
        h = t // NBCH
        bcol = b0 + (t % NBCH) * BCH
        pltpu.async_copy(
            trb_v.at[buf], out_hbm.at[h, :, pl.ds(bcol, BCH)], ssem.at[buf]
        )

    prep_and_fire(0, 0)

    def body(t2, _):
        for k in range(2):
            t = t2 * 2 + k
            buf = k
            if k == 0:
                prep_and_fire(t + 1, 1 - buf)
            else:
                @pl.when(t2 < NBLK // 2 - 1)
                def _():
                    prep_and_fire(t + 1, 1 - buf)
            pltpu.make_async_copy(
                packed_hbm.at[sidx_v.at[buf]], grp_v.at[buf], gsem
            ).wait()
            @pl.when(t2 >= 1)
            def _():
                pltpu.make_async_copy(
                    trb_v.at[buf], out_hbm.at[0, :, pl.ds(0, BCH)], ssem.at[buf]
                ).wait()
            extract(t, buf)
            write_out(t, buf)
        return 0

    lax.fori_loop(0, NBLK // 2, body, 0)

    for buf in range(2):
        pltpu.make_async_copy(
            trb_v.at[buf], out_hbm.at[0, :, pl.ds(0, BCH)], ssem.at[buf]
        ).wait()


def kernel(inputs, embeddings):
    packed = embeddings.reshape(VOCAB // 4, 128)
    idx_t = inputs.astype(jnp.int32).T
    out = _embed_sc(idx_t, packed)
    return out.transpose(2, 0, 1)


# speedup vs baseline: 1.8842x; 1.0078x over previous
import functools

import jax
import jax.numpy as jnp
from jax import lax
from jax.experimental import pallas as pl
from jax.experimental.pallas import tpu as pltpu
from jax.experimental.pallas import tpu_sc as plsc

BATCH = 16384
HIST = 50
VOCAB = 1000000
DIM = 32

NC = 2
NS = 16
NW = NC * NS
BPW = BATCH // NW
BCH = 128
NBCH = BPW // BCH
NBLK = HIST * NBCH

_MESH = plsc.VectorSubcoreMesh(core_axis_name="c", subcore_axis_name="s")

TBLK = 512
NTB = VOCAB // TBLK
TREM = VOCAB - NTB * TBLK
SLOTS = (NTB + NW - 1) // NW


@functools.partial(
    pl.kernel,
    mesh=_MESH,
    compiler_params=pltpu.CompilerParams(needs_layout_passes=False),
    out_type=jax.ShapeDtypeStruct((VOCAB // 4, 128), jnp.float32),
    scratch_types=[
        pltpu.VMEM((2, DIM, TBLK), jnp.float32),
        pltpu.VMEM((2, TBLK // 4, 128), jnp.float32),
        pltpu.VMEM((DIM, TREM), jnp.float32),
        pltpu.SemaphoreType.DMA((2,)),
        pltpu.SemaphoreType.DMA((2,)),
    ],
)
def _pack_sc(tT_hbm, packed_hbm, tin_v, tout_v, trem_v, gsem, ssem):
    wid = lax.axis_index("s") * NC + lax.axis_index("c")
    iota16 = lax.iota(jnp.int32, 16)

    def fire_in(j, buf):
        pltpu.async_copy(tT_hbm.at[:, pl.ds(j * TBLK, TBLK)], tin_v.at[buf], gsem.at[buf])

    def transpose_block(buf):
        def cg_body(cg, _):
            vbase = cg * 16
            lane = iota16 + vbase
            rows = lax.shift_right_logical(lane, 2)
            colb = jnp.bitwise_and(lane, 3) * 32
            for d8 in range(0, DIM, 8):
                vals = [tin_v[buf, d8 + i, pl.ds(vbase, 16)] for i in range(8)]
                for i in range(8):
                    plsc.store_scatter(
                        tout_v.at[buf], [rows, colb + (d8 + i)], vals[i]
                    )
            return 0

        lax.fori_loop(0, TBLK // 16, cg_body, 0)

    def fire_out(j, buf):
        pltpu.async_copy(
            tout_v.at[buf],
            packed_hbm.at[pl.ds(j * (TBLK // 4), TBLK // 4)],
            ssem.at[buf],
        )

    @pl.when(wid < NTB)
    def _():
        fire_in(wid, 0)

    def slot_body(k2, _):
        for b in range(2):
            slot = k2 * 2 + b
            j = wid + NW * slot
            jn = j + NW

            @pl.when(jn < NTB)
            def _():
                fire_in(jn, 1 - b)

            @pl.when(j < NTB)
            def _():
                pltpu.make_async_copy(
                    tT_hbm.at[:, pl.ds(0, TBLK)], tin_v.at[b], gsem.at[b]
                ).wait()

                @pl.when(slot >= 2)
                def _():
                    pltpu.make_async_copy(
                        tout_v.at[b],
                        packed_hbm.at[pl.ds(0, TBLK // 4)],
                        ssem.at[b],
                    ).wait()

                transpose_block(b)
                fire_out(j, b)
        return 0

    lax.fori_loop(0, SLOTS // 2, slot_body, 0)

    for b in range(2):
        nwr = (NTB - wid + NW - 1) // NW

        @pl.when(nwr >= b + 1)
        def _():
            pltpu.make_async_copy(
                tout_v.at[b], packed_hbm.at[pl.ds(0, TBLK // 4)], ssem.at[b]
            ).wait()

    @pl.when(wid == NW - 1)
    def _():
        pltpu.sync_copy(tT_hbm.at[:, pl.ds(NTB * TBLK, TREM)], trem_v)

        def cg_body(cg, _):
            vbase = cg * 16
            lane = iota16 + vbase
            rows = lax.shift_right_logical(lane, 2)
            colb = jnp.bitwise_and(lane, 3) * 32
            for d8 in range(0, DIM, 8):
                vals = [trem_v[d8 + i, pl.ds(vbase, 16)] for i in range(8)]
                for i in range(8):
                    plsc.store_scatter(
                        tout_v.at[0], [rows, colb + (d8 + i)], vals[i]
                    )
            return 0

        lax.fori_loop(0, TREM // 16, cg_body, 0)
        pltpu.sync_copy(
            tout_v.at[0, pl.ds(0, TREM // 4), :],
            packed_hbm.at[pl.ds(NTB * TBLK // 4, TREM // 4)],
        )


@functools.partial(
    pl.kernel,
    mesh=_MESH,
    compiler_params=pltpu.CompilerParams(needs_layout_passes=False),
    out_type=jax.ShapeDtypeStruct((HIST, DIM, BATCH), jnp.float32),
    scratch_types=[
        pltpu.VMEM((HIST, BPW), jnp.int32),
        pltpu.VMEM((2, BCH), jnp.int32),
        pltpu.VMEM((2, BCH, 128), jnp.float32),
        pltpu.VMEM((2, DIM, BCH), jnp.float32),
        pltpu.SemaphoreType.DMA,
        pltpu.SemaphoreType.DMA((2,)),
    ],
)
def _embed_sc(idx_hbm, packed_hbm, out_hbm, idx_v, sidx_v, grp_v, trb_v, gsem, ssem):
    wid = lax.axis_index("s") * NC + lax.axis_index("c")
    b0 = wid * BPW
    pltpu.sync_copy(idx_hbm.at[:, pl.ds(b0, BPW)], idx_v)

    iota16 = lax.iota(jnp.int32, 16)

    def prep_and_fire(t, buf):
        h = t // NBCH
        coff = (t % NBCH) * BCH
        for jv in range(BCH // 16):
            iv = idx_v[h, pl.ds(coff + jv * 16, 16)]
            sidx_v[buf, pl.ds(jv * 16, 16)] = lax.shift_right_logical(iv, 2)
        pltpu.async_copy(packed_hbm.at[sidx_v.at[buf]], grp_v.at[buf], gsem)

    def extract(t, buf):
        h = t // NBCH
        coff = (t % NBCH) * BCH
        for jv in range(BCH // 16):
            iv = idx_v[h, pl.ds(coff + jv * 16, 16)]
            colb = lax.shift_left(jnp.bitwise_and(iv, 3), 5)
            rows = iota16 + jv * 16
            for d8 in range(0, DIM, 8):
                vals = [
                    plsc.load_gather(grp_v.at[buf], [rows, colb + (d8 + i)])
                    for i in range(8)
                ]
                for i in range(8):
                    trb_v[buf, d8 + i, pl.ds(jv * 16, 16)] = vals[i]

    def write_out(t, buf):
        h = t // NBCH
        bcol = b0 + (t % NBCH) * BCH
        pltpu.async_copy(
            trb_v.at[buf], out_hbm.at[h, :, pl.ds(bcol, BCH)], ssem.at[buf]
        )

    prep_and_fire(0, 0)

    def body(t2, _):
        for k in range(2):
            t = t2 * 2 + k
            buf = k
            if k == 0:
                prep_and_fire(t + 1, 1 - buf)
            else:
                @pl.when(t2 < NBLK // 2 - 1)
                def _():
                    prep_and_fire(t + 1, 1 - buf)
            pltpu.make_async_copy(
                packed_hbm.at[sidx_v.at[buf]], grp_v.at[buf], gsem
            ).wait()
            @pl.when(t2 >= 1)
            def _():
                pltpu.make_async_copy(
                    trb_v.at[buf], out_hbm.at[0, :, pl.ds(0, BCH)], ssem.at[buf]
                ).wait()
            extract(t, buf)
            write_out(t, buf)
        return 0

    lax.fori_loop(0, NBLK // 2, body, 0)

    for buf in range(2):
        pltpu.make_async_copy(
            trb_v.at[buf], out_hbm.at[0, :, pl.ds(0, BCH)], ssem.at[buf]
        ).wait()


def kernel(inputs, embeddings):
    packed = _pack_sc(embeddings.T)
    idx_t = inputs.astype(jnp.int32).T
    out = _embed_sc(idx_t, packed)
    return out.transpose(2, 0, 1)
